# baseline (speedup 1.0000x reference)
import jax
import jax.numpy as jnp
from jax import lax
from jax.experimental import pallas as pl
from jax.experimental.pallas import tpu as pltpu
from jax.experimental.pallas import tpu_sc as plsc

B, N, D = 2, 2048, 1024
E, TOPK, DFF = 8, 2, 2 * 1024
T = B * N
TILE = 256
NT = T * TOPK // TILE + (E - 1)
PADT = NT * TILE
NC, NS, NL = 2, 16, 16
NW = NC * NS
TPW = T // NW
NCH = TPW // NL
DI = D // 2


def _router_body(x_ref, wg_ref, bg_ref,
                 i1_ref, i2_ref, r1_ref, r2_ref, g1_ref, g2_ref,
                 seg_ref, eot_ref, carry_ref):
    step = pl.program_id(0)

    @pl.when(step == 0)
    def _():
        carry_ref[...] = jnp.zeros_like(carry_ref)

    x = x_ref[...]
    logits = jnp.dot(x, wg_ref[...], preferred_element_type=jnp.float32) + bg_ref[...]
    col = lax.broadcasted_iota(jnp.int32, logits.shape, 1)
    l1 = jnp.max(logits, axis=1, keepdims=True)
    i1 = jnp.min(jnp.where(logits >= l1, col, E), axis=1, keepdims=True)
    masked = jnp.where(col == i1, -jnp.inf, logits)
    l2 = jnp.max(masked, axis=1, keepdims=True)
    i2 = jnp.min(jnp.where(masked >= l2, col, E), axis=1, keepdims=True)
    e2 = jnp.exp(l2 - l1)
    g1 = 1.0 / (1.0 + e2)
    g2 = e2 / (1.0 + e2)

    oh1 = (col == i1).astype(jnp.float32)
    oh2 = (col == i2).astype(jnp.float32)
    oh = oh1 + oh2
    r_i = lax.broadcasted_iota(jnp.int32, (TILE, TILE), 0)
    c_i = lax.broadcasted_iota(jnp.int32, (TILE, TILE), 1)
    tril = (r_i > c_i).astype(jnp.float32)
    carry = carry_ref[...]
    pre = jnp.dot(tril, oh, preferred_element_type=jnp.float32) + carry
    rank1 = jnp.sum(pre * oh1, axis=1, keepdims=True)
    rank2 = jnp.sum(pre * oh2, axis=1, keepdims=True)
    counts = carry + jnp.sum(oh, axis=0, keepdims=True)
    carry_ref[...] = counts

    i1_ref[...] = i1
    i2_ref[...] = i2
    r1_ref[...] = rank1.astype(jnp.int32)
    r2_ref[...] = rank2.astype(jnp.int32)
    g1_ref[...] = g1
    g2_ref[...] = g2

    tiles_i = (counts.astype(jnp.int32) + (TILE - 1)) // TILE
    triu = (r_i[:E, :E] < c_i[:E, :E]).astype(jnp.float32)
    base_t = jnp.dot(tiles_i.astype(jnp.float32), triu,
                     preferred_element_type=jnp.float32).astype(jnp.int32)
    seg_ref[...] = base_t * TILE
    ti = lax.broadcasted_iota(jnp.int32, (64, E), 0)
    ge = (ti >= base_t).astype(jnp.int32)
    eot_ref[...] = jnp.sum(ge, axis=1, keepdims=True) - 1


def _router_call(x_flat, Wg, bg2):
    n_blocks = T // TILE
    return pl.pallas_call(
        _router_body,
        grid=(n_blocks,),
        in_specs=[
            pl.BlockSpec((TILE, D), lambda i: (i, 0)),
            pl.BlockSpec((D, E), lambda i: (0, 0)),
            pl.BlockSpec((1, E), lambda i: (0, 0)),
        ],
        out_specs=[
            pl.BlockSpec((TILE, 1), lambda i: (i, 0)),
            pl.BlockSpec((TILE, 1), lambda i: (i, 0)),
            pl.BlockSpec((TILE, 1), lambda i: (i, 0)),
            pl.BlockSpec((TILE, 1), lambda i: (i, 0)),
            pl.BlockSpec((TILE, 1), lambda i: (i, 0)),
            pl.BlockSpec((TILE, 1), lambda i: (i, 0)),
            pl.BlockSpec((1, E), lambda i: (0, 0)),
            pl.BlockSpec((64, 1), lambda i: (0, 0)),
        ],
        out_shape=[
            jax.ShapeDtypeStruct((T, 1), jnp.int32),
            jax.ShapeDtypeStruct((T, 1), jnp.int32),
            jax.ShapeDtypeStruct((T, 1), jnp.int32),
            jax.ShapeDtypeStruct((T, 1), jnp.int32),
            jax.ShapeDtypeStruct((T, 1), jnp.float32),
            jax.ShapeDtypeStruct((T, 1), jnp.float32),
            jax.ShapeDtypeStruct((1, E), jnp.int32),
            jax.ShapeDtypeStruct((64, 1), jnp.int32),
        ],
        scratch_shapes=[pltpu.VMEM((1, E), jnp.float32)],
        compiler_params=pltpu.CompilerParams(
            dimension_semantics=("arbitrary",),
        ),
    )(x_flat, Wg, bg2)


_NBUF = 4


def _dispatch_body(x_hbm, i1_hbm, i2_hbm, r1_hbm, r2_hbm, seg_hbm,
                   xs_hbm, d1_hbm, d2_hbm,
                   i1_v, i2_v, r1_v, r2_v, seg_v, d1_v, d2_v, xbufs,
                   gsem, ssem):
    wid = lax.axis_index("s") * NC + lax.axis_index("c")
    base = wid * TPW
    pltpu.sync_copy(i1_hbm.at[pl.ds(base, TPW)], i1_v)
    pltpu.sync_copy(i2_hbm.at[pl.ds(base, TPW)], i2_v)
    pltpu.sync_copy(r1_hbm.at[pl.ds(base, TPW)], r1_v)
    pltpu.sync_copy(r2_hbm.at[pl.ds(base, TPW)], r2_v)
    pltpu.sync_copy(seg_hbm, seg_v)
    dests = []
    for c in range(NCH):
        i1c = i1_v[pl.ds(c * NL, NL)]
        i2c = i2_v[pl.ds(c * NL, NL)]
        d1 = plsc.load_gather(seg_v, [i1c]) + r1_v[pl.ds(c * NL, NL)]
        d2 = plsc.load_gather(seg_v, [i2c]) + r2_v[pl.ds(c * NL, NL)]
        d1_v[pl.ds(c * NL, NL)] = d1
        d2_v[pl.ds(c * NL, NL)] = d2
        dests.append((d1, d2))
    cpd1 = pltpu.async_copy(d1_v, d1_hbm.at[pl.ds(base, TPW)], ssem)
    cpd2 = pltpu.async_copy(d2_v, d2_hbm.at[pl.ds(base, TPW)], ssem)
    gcp = [None] * NCH
    scp = [None] * NCH
    for c in range(min(_NBUF, NCH)):
        gcp[c] = pltpu.async_copy(
            x_hbm.at[pl.ds(base + c * NL, NL), :], xbufs.at[c % _NBUF], gsem)
    for c in range(NCH):
        if c >= 1 and c + _NBUF - 1 < NCH:
            scp[c - 1][0].wait()
            scp[c - 1][1].wait()
            nxt = c + _NBUF - 1
            gcp[nxt] = pltpu.async_copy(
                x_hbm.at[pl.ds(base + nxt * NL, NL), :],
                xbufs.at[nxt % _NBUF], gsem)
        gcp[c].wait()
        d1, d2 = dests[c]
        s1 = pltpu.async_copy(xbufs.at[c % _NBUF], xs_hbm.at[d1], ssem)
        s2 = pltpu.async_copy(xbufs.at[c % _NBUF], xs_hbm.at[d2], ssem)
        scp[c] = (s1, s2)
    for c in range(max(0, NCH - _NBUF), NCH):
        scp[c][0].wait()
        scp[c][1].wait()
    cpd1.wait()
    cpd2.wait()


def _dispatch_call(x_flat, i1, i2, r1, r2, seg16):
    mesh = plsc.VectorSubcoreMesh(core_axis_name="c", subcore_axis_name="s")
    return pl.kernel(
        _dispatch_body,
        out_type=[
            jax.ShapeDtypeStruct((PADT, DI), jnp.int32),
            jax.ShapeDtypeStruct((T,), jnp.int32),
            jax.ShapeDtypeStruct((T,), jnp.int32),
        ],
        mesh=mesh,
        scratch_types=[
            pltpu.VMEM((TPW,), jnp.int32),
            pltpu.VMEM((TPW,), jnp.int32),
            pltpu.VMEM((TPW,), jnp.int32),
            pltpu.VMEM((TPW,), jnp.int32),
            pltpu.VMEM((NL,), jnp.int32),
            pltpu.VMEM((TPW,), jnp.int32),
            pltpu.VMEM((TPW,), jnp.int32),
            pltpu.VMEM((_NBUF, NL, DI), jnp.int32),
            pltpu.SemaphoreType.DMA,
            pltpu.SemaphoreType.DMA,
        ],
        compiler_params=pltpu.CompilerParams(needs_layout_passes=False),
    )(x_flat, i1, i2, r1, r2, seg16)


def _ffn_body(eot_ref, x_ref, w1_ref, b1_ref, w2_ref, b2_ref, y_ref):
    x = x_ref[...]
    h = jnp.dot(x, w1_ref[0], preferred_element_type=jnp.float32) + b1_ref[0]
    h = jnp.maximum(h, 0.0).astype(jnp.bfloat16)
    y = jnp.dot(h, w2_ref[0], preferred_element_type=jnp.float32) + b2_ref[0]
    y_ref[...] = y


def _ffn_call(eot, xs, W1, b1r, W2, b2r):
    grid_spec = pltpu.PrefetchScalarGridSpec(
        num_scalar_prefetch=1,
        grid=(NT,),
        in_specs=[
            pl.BlockSpec((TILE, D), lambda i, eot: (i, 0)),
            pl.BlockSpec((1, D, DFF), lambda i, eot: (eot[i], 0, 0)),
            pl.BlockSpec((1, 1, DFF), lambda i, eot: (eot[i], 0, 0)),
            pl.BlockSpec((1, DFF, D), lambda i, eot: (eot[i], 0, 0)),
            pl.BlockSpec((1, 1, D), lambda i, eot: (eot[i], 0, 0)),
        ],
        out_specs=pl.BlockSpec((TILE, D), lambda i, eot: (i, 0)),
    )
    return pl.pallas_call(
        _ffn_body,
        grid_spec=grid_spec,
        out_shape=jax.ShapeDtypeStruct((PADT, D), jnp.float32),
        compiler_params=pltpu.CompilerParams(
            dimension_semantics=("arbitrary",),
        ),
    )(eot, xs, W1, b1r, W2, b2r)


def _combine_body(y_hbm, d1_hbm, d2_hbm, g1_hbm, g2_hbm, out_hbm,
                  d1_v, d2_v, g1_v, g2_v, bufa, bufb, obuf, gsem, osem):
    wid = lax.axis_index("s") * NC + lax.axis_index("c")
    base = wid * TPW
    pltpu.sync_copy(d1_hbm.at[pl.ds(base, TPW)], d1_v)
    pltpu.sync_copy(d2_hbm.at[pl.ds(base, TPW)], d2_v)
    pltpu.sync_copy(g1_hbm.at[pl.ds(base, TPW)], g1_v)
    pltpu.sync_copy(g2_hbm.at[pl.ds(base, TPW)], g2_v)

    def issue_gathers(c):
        idx1 = d1_v[pl.ds(c * NL, NL)]
        idx2 = d2_v[pl.ds(c * NL, NL)]
        cp1 = pltpu.async_copy(y_hbm.at[idx1], bufa.at[c % 2], gsem)
        cp2 = pltpu.async_copy(y_hbm.at[idx2], bufb.at[c % 2], gsem)
        return cp1, cp2

    gcp = [None] * NCH
    ocp = [None] * NCH
    gcp[0] = issue_gathers(0)
    gcp[1] = issue_gathers(1)
    for c in range(NCH):
        gcp[c][0].wait()
        gcp[c][1].wait()
        if c >= 2:
            ocp[c - 2].wait()
        ba = bufa.at[c % 2]
        bb = bufb.at[c % 2]
        ob = obuf.at[c % 2]

        def row_body(r, _):
            ridx = jnp.full((NL,), c * NL + r, jnp.int32)
            ga = plsc.load_gather(g1_v, [ridx])
            gb = plsc.load_gather(g2_v, [ridx])
            for k in range(D // NL):
                a = ba[r, pl.ds(k * NL, NL)]
                b = bb[r, pl.ds(k * NL, NL)]
                ob[r, pl.ds(k * NL, NL)] = a * ga + b * gb
            return 0

        lax.fori_loop(0, NL, row_body, 0)
        if c + 2 < NCH:
            gcp[c + 2] = issue_gathers(c + 2)
        ocp[c] = pltpu.async_copy(ob, out_hbm.at[pl.ds(base + c * NL, NL), :],
                                  osem)
    ocp[NCH - 2].wait()
    ocp[NCH - 1].wait()


def _combine_call(ys, d1, d2, g1, g2):
    mesh = plsc.VectorSubcoreMesh(core_axis_name="c", subcore_axis_name="s")
    return pl.kernel(
        _combine_body,
        out_type=jax.ShapeDtypeStruct((T, D), jnp.float32),
        mesh=mesh,
        scratch_types=[
            pltpu.VMEM((TPW,), jnp.int32),
            pltpu.VMEM((TPW,), jnp.int32),
            pltpu.VMEM((TPW,), jnp.float32),
            pltpu.VMEM((TPW,), jnp.float32),
            pltpu.VMEM((2, NL, D), jnp.float32),
            pltpu.VMEM((2, NL, D), jnp.float32),
            pltpu.VMEM((2, NL, D), jnp.float32),
            pltpu.SemaphoreType.DMA,
            pltpu.SemaphoreType.DMA,
        ],
        compiler_params=pltpu.CompilerParams(needs_layout_passes=False),
    )(ys, d1, d2, g1, g2)


def kernel(x, W1, b1, W2, b2, Wg, bg):
    x_flat = x.reshape(T, D)
    (i1, i2, r1, r2, g1, g2, seg, eot) = _router_call(
        x_flat, Wg, bg.reshape(1, E))
    i1 = i1.reshape(T)
    i2 = i2.reshape(T)
    r1 = r1.reshape(T)
    r2 = r2.reshape(T)
    g1 = g1.reshape(T)
    g2 = g2.reshape(T)
    seg16 = jnp.pad(seg.reshape(E), (0, NL - E))
    eot = eot.reshape(64)[:NT]
    x_i32 = lax.bitcast_convert_type(
        x_flat.astype(jnp.bfloat16).reshape(T, DI, 2), jnp.int32)
    xs_i32, d1, d2 = _dispatch_call(x_i32, i1, i2, r1, r2, seg16)
    xs = lax.bitcast_convert_type(xs_i32, jnp.bfloat16).reshape(PADT, D)
    ys = _ffn_call(eot, xs, W1.astype(jnp.bfloat16), b1.reshape(E, 1, DFF),
                   W2.astype(jnp.bfloat16), b2.reshape(E, 1, D))
    out = _combine_call(ys, d1, d2, g1, g2)
    return out.reshape(B, N, D)


# speedup vs baseline: 2.1666x; 2.1666x over previous
import jax
import jax.numpy as jnp
from jax import lax
from jax.experimental import pallas as pl
from jax.experimental.pallas import tpu as pltpu
from jax.experimental.pallas import tpu_sc as plsc

B, N, D = 2, 2048, 1024
E, TOPK, DFF = 8, 2, 2 * 1024
T = B * N
TILE = 128
NT = T * TOPK // TILE + (E - 1)
PADT = NT * TILE
NC, NS, NL = 2, 16, 16
NW = NC * NS
TPW = T // NW
NCH = TPW // NL
DI = D // 2


def _router_body(x_ref, wg_ref, bg_ref,
                 i1_ref, i2_ref, r1_ref, r2_ref, g1_ref, g2_ref,
                 seg_ref, eot_ref, carry_ref):
    step = pl.program_id(0)

    @pl.when(step == 0)
    def _():
        carry_ref[...] = jnp.zeros_like(carry_ref)

    x = x_ref[...]
    logits = jnp.dot(x, wg_ref[...], preferred_element_type=jnp.float32) + bg_ref[...]
    col = lax.broadcasted_iota(jnp.int32, logits.shape, 1)
    l1 = jnp.max(logits, axis=1, keepdims=True)
    i1 = jnp.min(jnp.where(logits >= l1, col, E), axis=1, keepdims=True)
    masked = jnp.where(col == i1, -jnp.inf, logits)
    l2 = jnp.max(masked, axis=1, keepdims=True)
    i2 = jnp.min(jnp.where(masked >= l2, col, E), axis=1, keepdims=True)
    e2 = jnp.exp(l2 - l1)
    g1 = 1.0 / (1.0 + e2)
    g2 = e2 / (1.0 + e2)

    oh1 = (col == i1).astype(jnp.float32)
    oh2 = (col == i2).astype(jnp.float32)
    oh = oh1 + oh2
    r_i = lax.broadcasted_iota(jnp.int32, (TILE, TILE), 0)
    c_i = lax.broadcasted_iota(jnp.int32, (TILE, TILE), 1)
    tril = (r_i > c_i).astype(jnp.float32)
    carry = carry_ref[...]
    pre = jnp.dot(tril, oh, preferred_element_type=jnp.float32) + carry
    rank1 = jnp.sum(pre * oh1, axis=1, keepdims=True)
    rank2 = jnp.sum(pre * oh2, axis=1, keepdims=True)
    counts = carry + jnp.sum(oh, axis=0, keepdims=True)
    carry_ref[...] = counts

    i1_ref[...] = i1
    i2_ref[...] = i2
    r1_ref[...] = rank1.astype(jnp.int32)
    r2_ref[...] = rank2.astype(jnp.int32)
    g1_ref[...] = g1
    g2_ref[...] = g2

    tiles_i = (counts.astype(jnp.int32) + (TILE - 1)) // TILE
    triu = (r_i[:E, :E] < c_i[:E, :E]).astype(jnp.float32)
    base_t = jnp.dot(tiles_i.astype(jnp.float32), triu,
                     preferred_element_type=jnp.float32).astype(jnp.int32)
    seg_ref[...] = base_t * TILE
    ti = lax.broadcasted_iota(jnp.int32, (128, E), 0)
    ge = (ti >= base_t).astype(jnp.int32)
    eot_ref[...] = jnp.sum(ge, axis=1, keepdims=True) - 1


def _router_call(x_flat, Wg, bg2):
    n_blocks = T // TILE
    return pl.pallas_call(
        _router_body,
        grid=(n_blocks,),
        in_specs=[
            pl.BlockSpec((TILE, D), lambda i: (i, 0)),
            pl.BlockSpec((D, E), lambda i: (0, 0)),
            pl.BlockSpec((1, E), lambda i: (0, 0)),
        ],
        out_specs=[
            pl.BlockSpec((TILE, 1), lambda i: (i, 0)),
            pl.BlockSpec((TILE, 1), lambda i: (i, 0)),
            pl.BlockSpec((TILE, 1), lambda i: (i, 0)),
            pl.BlockSpec((TILE, 1), lambda i: (i, 0)),
            pl.BlockSpec((TILE, 1), lambda i: (i, 0)),
            pl.BlockSpec((TILE, 1), lambda i: (i, 0)),
            pl.BlockSpec((1, E), lambda i: (0, 0)),
            pl.BlockSpec((128, 1), lambda i: (0, 0)),
        ],
        out_shape=[
            jax.ShapeDtypeStruct((T, 1), jnp.int32),
            jax.ShapeDtypeStruct((T, 1), jnp.int32),
            jax.ShapeDtypeStruct((T, 1), jnp.int32),
            jax.ShapeDtypeStruct((T, 1), jnp.int32),
            jax.ShapeDtypeStruct((T, 1), jnp.float32),
            jax.ShapeDtypeStruct((T, 1), jnp.float32),
            jax.ShapeDtypeStruct((1, E), jnp.int32),
            jax.ShapeDtypeStruct((128, 1), jnp.int32),
        ],
        scratch_shapes=[pltpu.VMEM((1, E), jnp.float32)],
        compiler_params=pltpu.CompilerParams(
            dimension_semantics=("arbitrary",),
        ),
    )(x_flat, Wg, bg2)


_NBUF = 4


def _dispatch_body(x_hbm, i1_hbm, i2_hbm, r1_hbm, r2_hbm, seg_hbm,
                   xs_hbm, d1_hbm, d2_hbm,
                   i1_v, i2_v, r1_v, r2_v, seg_v, d1_v, d2_v, xbufs,
                   gsem, ssem):
    wid = lax.axis_index("s") * NC + lax.axis_index("c")
    base = wid * TPW
    pltpu.sync_copy(i1_hbm.at[pl.ds(base, TPW)], i1_v)
    pltpu.sync_copy(i2_hbm.at[pl.ds(base, TPW)], i2_v)
    pltpu.sync_copy(r1_hbm.at[pl.ds(base, TPW)], r1_v)
    pltpu.sync_copy(r2_hbm.at[pl.ds(base, TPW)], r2_v)
    pltpu.sync_copy(seg_hbm, seg_v)
    dests = []
    for c in range(NCH):
        i1c = i1_v[pl.ds(c * NL, NL)]
        i2c = i2_v[pl.ds(c * NL, NL)]
        d1 = plsc.load_gather(seg_v, [i1c]) + r1_v[pl.ds(c * NL, NL)]
        d2 = plsc.load_gather(seg_v, [i2c]) + r2_v[pl.ds(c * NL, NL)]
        d1_v[pl.ds(c * NL, NL)] = d1
        d2_v[pl.ds(c * NL, NL)] = d2
        dests.append((d1, d2))
    cpd1 = pltpu.async_copy(d1_v, d1_hbm.at[pl.ds(base, TPW)], ssem)
    cpd2 = pltpu.async_copy(d2_v, d2_hbm.at[pl.ds(base, TPW)], ssem)
    gcp = [None] * NCH
    scp = [None] * NCH
    for c in range(min(_NBUF, NCH)):
        gcp[c] = pltpu.async_copy(
            x_hbm.at[pl.ds(base + c * NL, NL), :], xbufs.at[c % _NBUF], gsem)
    for c in range(NCH):
        if c >= 1 and c + _NBUF - 1 < NCH:
            scp[c - 1][0].wait()
            scp[c - 1][1].wait()
            nxt = c + _NBUF - 1
            gcp[nxt] = pltpu.async_copy(
                x_hbm.at[pl.ds(base + nxt * NL, NL), :],
                xbufs.at[nxt % _NBUF], gsem)
        gcp[c].wait()
        d1, d2 = dests[c]
        s1 = pltpu.async_copy(xbufs.at[c % _NBUF], xs_hbm.at[d1], ssem)
        s2 = pltpu.async_copy(xbufs.at[c % _NBUF], xs_hbm.at[d2], ssem)
        scp[c] = (s1, s2)
    for c in range(max(0, NCH - _NBUF), NCH):
        scp[c][0].wait()
        scp[c][1].wait()
    cpd1.wait()
    cpd2.wait()


def _dispatch_call(x_flat, i1, i2, r1, r2, seg16):
    mesh = plsc.VectorSubcoreMesh(core_axis_name="c", subcore_axis_name="s")
    return pl.kernel(
        _dispatch_body,
        out_type=[
            jax.ShapeDtypeStruct((PADT, D), jnp.float32),
            jax.ShapeDtypeStruct((T,), jnp.int32),
            jax.ShapeDtypeStruct((T,), jnp.int32),
        ],
        mesh=mesh,
        scratch_types=[
            pltpu.VMEM((TPW,), jnp.int32),
            pltpu.VMEM((TPW,), jnp.int32),
            pltpu.VMEM((TPW,), jnp.int32),
            pltpu.VMEM((TPW,), jnp.int32),
            pltpu.VMEM((NL,), jnp.int32),
            pltpu.VMEM((TPW,), jnp.int32),
            pltpu.VMEM((TPW,), jnp.int32),
            pltpu.VMEM((_NBUF, NL, D), jnp.float32),
            pltpu.SemaphoreType.DMA,
            pltpu.SemaphoreType.DMA,
        ],
        compiler_params=pltpu.CompilerParams(needs_layout_passes=False),
    )(x_flat, i1, i2, r1, r2, seg16)


def _ffn_body(eot_ref, x_ref, w1_ref, b1_ref, w2_ref, b2_ref, y_ref):
    x = x_ref[...]
    h = jnp.dot(x, w1_ref[0], preferred_element_type=jnp.float32) + b1_ref[0]
    h = jnp.maximum(h, 0.0)
    y = jnp.dot(h, w2_ref[0], preferred_element_type=jnp.float32) + b2_ref[0]
    y_ref[...] = y


def _ffn_call(eot, xs, W1, b1r, W2, b2r):
    grid_spec = pltpu.PrefetchScalarGridSpec(
        num_scalar_prefetch=1,
        grid=(NT,),
        in_specs=[
            pl.BlockSpec((TILE, D), lambda i, eot: (i, 0)),
            pl.BlockSpec((1, D, DFF), lambda i, eot: (eot[i], 0, 0)),
            pl.BlockSpec((1, 1, DFF), lambda i, eot: (eot[i], 0, 0)),
            pl.BlockSpec((1, DFF, D), lambda i, eot: (eot[i], 0, 0)),
            pl.BlockSpec((1, 1, D), lambda i, eot: (eot[i], 0, 0)),
        ],
        out_specs=pl.BlockSpec((TILE, D), lambda i, eot: (i, 0)),
    )
    return pl.pallas_call(
        _ffn_body,
        grid_spec=grid_spec,
        out_shape=jax.ShapeDtypeStruct((PADT, D), jnp.float32),
        compiler_params=pltpu.CompilerParams(
            dimension_semantics=("arbitrary",),
        ),
    )(eot, xs, W1, b1r, W2, b2r)


def _combine_body(y_hbm, d1_hbm, d2_hbm, g1_hbm, g2_hbm, out_hbm,
                  d1_v, d2_v, g1_v, g2_v, bufa, bufb, obuf, gsem, osem):
    wid = lax.axis_index("s") * NC + lax.axis_index("c")
    base = wid * TPW
    pltpu.sync_copy(d1_hbm.at[pl.ds(base, TPW)], d1_v)
    pltpu.sync_copy(d2_hbm.at[pl.ds(base, TPW)], d2_v)
    pltpu.sync_copy(g1_hbm.at[pl.ds(base, TPW)], g1_v)
    pltpu.sync_copy(g2_hbm.at[pl.ds(base, TPW)], g2_v)

    def issue_gathers(c):
        idx1 = d1_v[pl.ds(c * NL, NL)]
        idx2 = d2_v[pl.ds(c * NL, NL)]
        cp1 = pltpu.async_copy(y_hbm.at[idx1], bufa.at[c % 2], gsem)
        cp2 = pltpu.async_copy(y_hbm.at[idx2], bufb.at[c % 2], gsem)
        return cp1, cp2

    gcp = [None] * NCH
    ocp = [None] * NCH
    gcp[0] = issue_gathers(0)
    gcp[1] = issue_gathers(1)
    for c in range(NCH):
        gcp[c][0].wait()
        gcp[c][1].wait()
        if c >= 2:
            ocp[c - 2].wait()
        ba = bufa.at[c % 2]
        bb = bufb.at[c % 2]
        ob = obuf.at[c % 2]

        def row_body(r, _):
            ridx = jnp.full((NL,), c * NL + r, jnp.int32)
            ga = plsc.load_gather(g1_v, [ridx])
            gb = plsc.load_gather(g2_v, [ridx])
            for k in range(D // NL):
                a = ba[r, pl.ds(k * NL, NL)]
                b = bb[r, pl.ds(k * NL, NL)]
                ob[r, pl.ds(k * NL, NL)] = a * ga + b * gb
            return 0

        lax.fori_loop(0, NL, row_body, 0)
        if c + 2 < NCH:
            gcp[c + 2] = issue_gathers(c + 2)
        ocp[c] = pltpu.async_copy(ob, out_hbm.at[pl.ds(base + c * NL, NL), :],
                                  osem)
    ocp[NCH - 2].wait()
    ocp[NCH - 1].wait()


def _combine_call(ys, d1, d2, g1, g2):
    mesh = plsc.VectorSubcoreMesh(core_axis_name="c", subcore_axis_name="s")
    return pl.kernel(
        _combine_body,
        out_type=jax.ShapeDtypeStruct((T, D), jnp.float32),
        mesh=mesh,
        scratch_types=[
            pltpu.VMEM((TPW,), jnp.int32),
            pltpu.VMEM((TPW,), jnp.int32),
            pltpu.VMEM((TPW,), jnp.float32),
            pltpu.VMEM((TPW,), jnp.float32),
            pltpu.VMEM((2, NL, D), jnp.float32),
            pltpu.VMEM((2, NL, D), jnp.float32),
            pltpu.VMEM((2, NL, D), jnp.float32),
            pltpu.SemaphoreType.DMA,
            pltpu.SemaphoreType.DMA,
        ],
        compiler_params=pltpu.CompilerParams(needs_layout_passes=False),
    )(ys, d1, d2, g1, g2)


def kernel(x, W1, b1, W2, b2, Wg, bg):
    x_flat = x.reshape(T, D)
    (i1, i2, r1, r2, g1, g2, seg, eot) = _router_call(
        x_flat, Wg, bg.reshape(1, E))
    i1 = i1.reshape(T)
    i2 = i2.reshape(T)
    r1 = r1.reshape(T)
    r2 = r2.reshape(T)
    g1 = g1.reshape(T)
    g2 = g2.reshape(T)
    seg16 = jnp.pad(seg.reshape(E), (0, NL - E))
    eot = eot.reshape(128)[:NT]
    xs, d1, d2 = _dispatch_call(x_flat, i1, i2, r1, r2, seg16)
    ys = _ffn_call(eot, xs, W1, b1.reshape(E, 1, DFF), W2, b2.reshape(E, 1, D))
    out = _combine_call(ys, d1, d2, g1, g2)
    return out.reshape(B, N, D)


# speedup vs baseline: 2.3965x; 1.1061x over previous
import jax
import jax.numpy as jnp
from jax import lax
from jax.experimental import pallas as pl
from jax.experimental.pallas import tpu as pltpu
from jax.experimental.pallas import tpu_sc as plsc

B, N, D = 2, 2048, 1024
E, TOPK, DFF = 8, 2, 2 * 1024
T = B * N
TILE = 256
NT = T * TOPK // TILE + (E - 1)
PADT = NT * TILE
NC, NS, NL = 2, 16, 16
NW = NC * NS
TPW = T // NW
NCH = TPW // NL
DI = D // 2


def _router_body(x_ref, wg_ref, bg_ref,
                 i1_ref, i2_ref, r1_ref, r2_ref, g1_ref, g2_ref,
                 seg_ref, eot_ref, carry_ref):
    step = pl.program_id(0)

    @pl.when(step == 0)
    def _():
        carry_ref[...] = jnp.zeros_like(carry_ref)

    x = x_ref[...]
    logits = jnp.dot(x, wg_ref[...], preferred_element_type=jnp.float32) + bg_ref[...]
    col = lax.broadcasted_iota(jnp.int32, logits.shape, 1)
    l1 = jnp.max(logits, axis=1, keepdims=True)
    i1 = jnp.min(jnp.where(logits >= l1, col, E), axis=1, keepdims=True)
    masked = jnp.where(col == i1, -jnp.inf, logits)
    l2 = jnp.max(masked, axis=1, keepdims=True)
    i2 = jnp.min(jnp.where(masked >= l2, col, E), axis=1, keepdims=True)
    e2 = jnp.exp(l2 - l1)
    g1 = 1.0 / (1.0 + e2)
    g2 = e2 / (1.0 + e2)

    oh1 = (col == i1).astype(jnp.float32)
    oh2 = (col == i2).astype(jnp.float32)
    oh = oh1 + oh2
    r_i = lax.broadcasted_iota(jnp.int32, (TILE, TILE), 0)
    c_i = lax.broadcasted_iota(jnp.int32, (TILE, TILE), 1)
    tril = (r_i > c_i).astype(jnp.float32)
    carry = carry_ref[...]
    pre = jnp.dot(tril, oh, preferred_element_type=jnp.float32) + carry
    rank1 = jnp.sum(pre * oh1, axis=1, keepdims=True)
    rank2 = jnp.sum(pre * oh2, axis=1, keepdims=True)
    counts = carry + jnp.sum(oh, axis=0, keepdims=True)
    carry_ref[...] = counts

    i1_ref[...] = i1
    i2_ref[...] = i2
    r1_ref[...] = rank1.astype(jnp.int32)
    r2_ref[...] = rank2.astype(jnp.int32)
    g1_ref[...] = g1
    g2_ref[...] = g2

    tiles_i = (counts.astype(jnp.int32) + (TILE - 1)) // TILE
    triu = (r_i[:E, :E] < c_i[:E, :E]).astype(jnp.float32)
    base_t = jnp.dot(tiles_i.astype(jnp.float32), triu,
                     preferred_element_type=jnp.float32).astype(jnp.int32)
    seg_ref[...] = base_t * TILE
    ti = lax.broadcasted_iota(jnp.int32, (128, E), 0)
    ge = (ti >= base_t).astype(jnp.int32)
    eot_ref[...] = jnp.sum(ge, axis=1, keepdims=True) - 1


def _router_call(x_flat, Wg, bg2):
    n_blocks = T // TILE
    return pl.pallas_call(
        _router_body,
        grid=(n_blocks,),
        in_specs=[
            pl.BlockSpec((TILE, D), lambda i: (i, 0)),
            pl.BlockSpec((D, E), lambda i: (0, 0)),
            pl.BlockSpec((1, E), lambda i: (0, 0)),
        ],
        out_specs=[
            pl.BlockSpec((TILE, 1), lambda i: (i, 0)),
            pl.BlockSpec((TILE, 1), lambda i: (i, 0)),
            pl.BlockSpec((TILE, 1), lambda i: (i, 0)),
            pl.BlockSpec((TILE, 1), lambda i: (i, 0)),
            pl.BlockSpec((TILE, 1), lambda i: (i, 0)),
            pl.BlockSpec((TILE, 1), lambda i: (i, 0)),
            pl.BlockSpec((1, E), lambda i: (0, 0)),
            pl.BlockSpec((128, 1), lambda i: (0, 0)),
        ],
        out_shape=[
            jax.ShapeDtypeStruct((T, 1), jnp.int32),
            jax.ShapeDtypeStruct((T, 1), jnp.int32),
            jax.ShapeDtypeStruct((T, 1), jnp.int32),
            jax.ShapeDtypeStruct((T, 1), jnp.int32),
            jax.ShapeDtypeStruct((T, 1), jnp.float32),
            jax.ShapeDtypeStruct((T, 1), jnp.float32),
            jax.ShapeDtypeStruct((1, E), jnp.int32),
            jax.ShapeDtypeStruct((128, 1), jnp.int32),
        ],
        scratch_shapes=[pltpu.VMEM((1, E), jnp.float32)],
        compiler_params=pltpu.CompilerParams(
            dimension_semantics=("arbitrary",),
        ),
    )(x_flat, Wg, bg2)


_NBUF = 4


def _dispatch_body(x_hbm, i1_hbm, i2_hbm, r1_hbm, r2_hbm, seg_hbm,
                   xs_hbm, d1_hbm, d2_hbm,
                   i1_v, i2_v, r1_v, r2_v, seg_v, d1_v, d2_v, xbufs,
                   gsem, ssem):
    wid = lax.axis_index("s") * NC + lax.axis_index("c")
    base = wid * TPW
    pltpu.sync_copy(i1_hbm.at[pl.ds(base, TPW)], i1_v)
    pltpu.sync_copy(i2_hbm.at[pl.ds(base, TPW)], i2_v)
    pltpu.sync_copy(r1_hbm.at[pl.ds(base, TPW)], r1_v)
    pltpu.sync_copy(r2_hbm.at[pl.ds(base, TPW)], r2_v)
    pltpu.sync_copy(seg_hbm, seg_v)
    dests = []
    for c in range(NCH):
        i1c = i1_v[pl.ds(c * NL, NL)]
        i2c = i2_v[pl.ds(c * NL, NL)]
        d1 = plsc.load_gather(seg_v, [i1c]) + r1_v[pl.ds(c * NL, NL)]
        d2 = plsc.load_gather(seg_v, [i2c]) + r2_v[pl.ds(c * NL, NL)]
        d1_v[pl.ds(c * NL, NL)] = d1
        d2_v[pl.ds(c * NL, NL)] = d2
        dests.append((d1, d2))
    cpd1 = pltpu.async_copy(d1_v, d1_hbm.at[pl.ds(base, TPW)], ssem)
    cpd2 = pltpu.async_copy(d2_v, d2_hbm.at[pl.ds(base, TPW)], ssem)
    gcp = [None] * NCH
    scp = [None] * NCH
    for c in range(min(_NBUF, NCH)):
        gcp[c] = pltpu.async_copy(
            x_hbm.at[pl.ds(base + c * NL, NL), :], xbufs.at[c % _NBUF], gsem)
    for c in range(NCH):
        if c >= 1 and c + _NBUF - 1 < NCH:
            scp[c - 1][0].wait()
            scp[c - 1][1].wait()
            nxt = c + _NBUF - 1
            gcp[nxt] = pltpu.async_copy(
                x_hbm.at[pl.ds(base + nxt * NL, NL), :],
                xbufs.at[nxt % _NBUF], gsem)
        gcp[c].wait()
        d1, d2 = dests[c]
        s1 = pltpu.async_copy(xbufs.at[c % _NBUF], xs_hbm.at[d1], ssem)
        s2 = pltpu.async_copy(xbufs.at[c % _NBUF], xs_hbm.at[d2], ssem)
        scp[c] = (s1, s2)
    for c in range(max(0, NCH - _NBUF), NCH):
        scp[c][0].wait()
        scp[c][1].wait()
    cpd1.wait()
    cpd2.wait()


def _dispatch_call(x_flat, i1, i2, r1, r2, seg16):
    mesh = plsc.VectorSubcoreMesh(core_axis_name="c", subcore_axis_name="s")
    return pl.kernel(
        _dispatch_body,
        out_type=[
            jax.ShapeDtypeStruct((PADT, D), jnp.float32),
            jax.ShapeDtypeStruct((T,), jnp.int32),
            jax.ShapeDtypeStruct((T,), jnp.int32),
        ],
        mesh=mesh,
        scratch_types=[
            pltpu.VMEM((TPW,), jnp.int32),
            pltpu.VMEM((TPW,), jnp.int32),
            pltpu.VMEM((TPW,), jnp.int32),
            pltpu.VMEM((TPW,), jnp.int32),
            pltpu.VMEM((NL,), jnp.int32),
            pltpu.VMEM((TPW,), jnp.int32),
            pltpu.VMEM((TPW,), jnp.int32),
            pltpu.VMEM((_NBUF, NL, D), jnp.float32),
            pltpu.SemaphoreType.DMA,
            pltpu.SemaphoreType.DMA,
        ],
        compiler_params=pltpu.CompilerParams(needs_layout_passes=False),
    )(x_flat, i1, i2, r1, r2, seg16)


def _ffn_body(eot_ref, x_ref, w1_ref, b1_ref, w2_ref, b2_ref, y_ref):
    x = x_ref[...]
    h = jnp.dot(x, w1_ref[0], preferred_element_type=jnp.float32) + b1_ref[0]
    h = jnp.maximum(h, 0.0)
    y = jnp.dot(h, w2_ref[0], preferred_element_type=jnp.float32) + b2_ref[0]
    y_ref[...] = y


def _ffn_call(eot, xs, W1, b1r, W2, b2r):
    grid_spec = pltpu.PrefetchScalarGridSpec(
        num_scalar_prefetch=1,
        grid=(NT,),
        in_specs=[
            pl.BlockSpec((TILE, D), lambda i, eot: (i, 0)),
            pl.BlockSpec((1, D, DFF), lambda i, eot: (eot[i], 0, 0)),
            pl.BlockSpec((1, 1, DFF), lambda i, eot: (eot[i], 0, 0)),
            pl.BlockSpec((1, DFF, D), lambda i, eot: (eot[i], 0, 0)),
            pl.BlockSpec((1, 1, D), lambda i, eot: (eot[i], 0, 0)),
        ],
        out_specs=pl.BlockSpec((TILE, D), lambda i, eot: (i, 0)),
    )
    return pl.pallas_call(
        _ffn_body,
        grid_spec=grid_spec,
        out_shape=jax.ShapeDtypeStruct((PADT, D), jnp.float32),
        compiler_params=pltpu.CompilerParams(
            dimension_semantics=("arbitrary",),
        ),
    )(eot, xs, W1, b1r, W2, b2r)


def _combine_body(y_hbm, d1_hbm, d2_hbm, g1_hbm, g2_hbm, out_hbm,
                  d1_v, d2_v, g1_v, g2_v, bufa, bufb, obuf, gsem, osem):
    wid = lax.axis_index("s") * NC + lax.axis_index("c")
    base = wid * TPW
    pltpu.sync_copy(d1_hbm.at[pl.ds(base, TPW)], d1_v)
    pltpu.sync_copy(d2_hbm.at[pl.ds(base, TPW)], d2_v)
    pltpu.sync_copy(g1_hbm.at[pl.ds(base, TPW)], g1_v)
    pltpu.sync_copy(g2_hbm.at[pl.ds(base, TPW)], g2_v)

    def issue_gathers(c):
        idx1 = d1_v[pl.ds(c * NL, NL)]
        idx2 = d2_v[pl.ds(c * NL, NL)]
        cp1 = pltpu.async_copy(y_hbm.at[idx1], bufa.at[c % 2], gsem)
        cp2 = pltpu.async_copy(y_hbm.at[idx2], bufb.at[c % 2], gsem)
        return cp1, cp2

    gcp = [None] * NCH
    ocp = [None] * NCH
    gcp[0] = issue_gathers(0)
    gcp[1] = issue_gathers(1)
    for c in range(NCH):
        gcp[c][0].wait()
        gcp[c][1].wait()
        if c >= 2:
            ocp[c - 2].wait()
        ba = bufa.at[c % 2]
        bb = bufb.at[c % 2]
        ob = obuf.at[c % 2]

        def row_body(r, _):
            ridx = jnp.full((NL,), c * NL + r, jnp.int32)
            ga = plsc.load_gather(g1_v, [ridx])
            gb = plsc.load_gather(g2_v, [ridx])
            for k in range(D // NL):
                a = ba[r, pl.ds(k * NL, NL)]
                b = bb[r, pl.ds(k * NL, NL)]
                ob[r, pl.ds(k * NL, NL)] = a * ga + b * gb
            return 0

        lax.fori_loop(0, NL, row_body, 0)
        if c + 2 < NCH:
            gcp[c + 2] = issue_gathers(c + 2)
        ocp[c] = pltpu.async_copy(ob, out_hbm.at[pl.ds(base + c * NL, NL), :],
                                  osem)
    ocp[NCH - 2].wait()
    ocp[NCH - 1].wait()


def _combine_call(ys, d1, d2, g1, g2):
    mesh = plsc.VectorSubcoreMesh(core_axis_name="c", subcore_axis_name="s")
    return pl.kernel(
        _combine_body,
        out_type=jax.ShapeDtypeStruct((T, D), jnp.float32),
        mesh=mesh,
        scratch_types=[
            pltpu.VMEM((TPW,), jnp.int32),
            pltpu.VMEM((TPW,), jnp.int32),
            pltpu.VMEM((TPW,), jnp.float32),
            pltpu.VMEM((TPW,), jnp.float32),
            pltpu.VMEM((2, NL, D), jnp.float32),
            pltpu.VMEM((2, NL, D), jnp.float32),
            pltpu.VMEM((2, NL, D), jnp.float32),
            pltpu.SemaphoreType.DMA,
            pltpu.SemaphoreType.DMA,
        ],
        compiler_params=pltpu.CompilerParams(needs_layout_passes=False),
    )(ys, d1, d2, g1, g2)


def kernel(x, W1, b1, W2, b2, Wg, bg):
    x_flat = x.reshape(T, D)
    (i1, i2, r1, r2, g1, g2, seg, eot) = _router_call(
        x_flat, Wg, bg.reshape(1, E))
    i1 = i1.reshape(T)
    i2 = i2.reshape(T)
    r1 = r1.reshape(T)
    r2 = r2.reshape(T)
    g1 = g1.reshape(T)
    g2 = g2.reshape(T)
    seg16 = jnp.pad(seg.reshape(E), (0, NL - E))
    eot = eot.reshape(128)[:NT]
    xs, d1, d2 = _dispatch_call(x_flat, i1, i2, r1, r2, seg16)
    ys = _ffn_call(eot, xs, W1, b1.reshape(E, 1, DFF), W2, b2.reshape(E, 1, D))
    out = _combine_call(ys, d1, d2, g1, g2)
    return out.reshape(B, N, D)


# speedup vs baseline: 2.5384x; 1.0592x over previous
import jax
import jax.numpy as jnp
from jax import lax
from jax.experimental import pallas as pl
from jax.experimental.pallas import tpu as pltpu
from jax.experimental.pallas import tpu_sc as plsc

B, N, D = 2, 2048, 1024
E, TOPK, DFF = 8, 2, 2 * 1024
T = B * N
TILE = 256
NT = T * TOPK // TILE + (E - 1)
PADT = NT * TILE
NC, NS, NL = 2, 16, 16
NW = NC * NS
TPW = T // NW
NCH = TPW // NL
DI = D // 2


def _router_body(x_ref, wg_ref, bg_ref,
                 i1_ref, i2_ref, r1_ref, r2_ref, g1_ref, g2_ref,
                 seg_ref, eot_ref, nxt_ref, slt_ref, carry_ref):
    step = pl.program_id(0)

    @pl.when(step == 0)
    def _():
        carry_ref[...] = jnp.zeros_like(carry_ref)

    x = x_ref[...]
    logits = jnp.dot(x, wg_ref[...], preferred_element_type=jnp.float32) + bg_ref[...]
    col = lax.broadcasted_iota(jnp.int32, logits.shape, 1)
    l1 = jnp.max(logits, axis=1, keepdims=True)
    i1 = jnp.min(jnp.where(logits >= l1, col, E), axis=1, keepdims=True)
    masked = jnp.where(col == i1, -jnp.inf, logits)
    l2 = jnp.max(masked, axis=1, keepdims=True)
    i2 = jnp.min(jnp.where(masked >= l2, col, E), axis=1, keepdims=True)
    e2 = jnp.exp(l2 - l1)
    g1 = 1.0 / (1.0 + e2)
    g2 = e2 / (1.0 + e2)

    oh1 = (col == i1).astype(jnp.float32)
    oh2 = (col == i2).astype(jnp.float32)
    oh = oh1 + oh2
    r_i = lax.broadcasted_iota(jnp.int32, (TILE, TILE), 0)
    c_i = lax.broadcasted_iota(jnp.int32, (TILE, TILE), 1)
    tril = (r_i > c_i).astype(jnp.float32)
    carry = carry_ref[...]
    pre = jnp.dot(tril, oh, preferred_element_type=jnp.float32) + carry
    rank1 = jnp.sum(pre * oh1, axis=1, keepdims=True)
    rank2 = jnp.sum(pre * oh2, axis=1, keepdims=True)
    counts = carry + jnp.sum(oh, axis=0, keepdims=True)
    carry_ref[...] = counts

    i1_ref[...] = i1
    i2_ref[...] = i2
    r1_ref[...] = rank1.astype(jnp.int32)
    r2_ref[...] = rank2.astype(jnp.int32)
    g1_ref[...] = g1
    g2_ref[...] = g2

    tiles_i = (counts.astype(jnp.int32) + (TILE - 1)) // TILE
    triu = (r_i[:E, :E] < c_i[:E, :E]).astype(jnp.float32)
    base_t = jnp.dot(tiles_i.astype(jnp.float32), triu,
                     preferred_element_type=jnp.float32).astype(jnp.int32)
    seg_ref[...] = base_t * TILE
    ti = lax.broadcasted_iota(jnp.int32, (128, E), 0)
    ge = (ti >= base_t).astype(jnp.int32)
    eot_ref[...] = jnp.sum(ge, axis=1, keepdims=True) - 1
    ecol = lax.broadcasted_iota(jnp.int32, (128, E), 1)
    nz = (tiles_i > 0)
    nxt_ref[...] = jnp.min(
        jnp.where(nz & (base_t > ti), ecol, E), axis=1, keepdims=True)
    ap = jnp.sum((nz & (base_t <= ti)).astype(jnp.int32), axis=1, keepdims=True)
    slt_ref[...] = (ap - 1) & 1


def _router_call(x_flat, Wg, bg2):
    n_blocks = T // TILE
    return pl.pallas_call(
        _router_body,
        grid=(n_blocks,),
        in_specs=[
            pl.BlockSpec((TILE, D), lambda i: (i, 0)),
            pl.BlockSpec((D, E), lambda i: (0, 0)),
            pl.BlockSpec((1, E), lambda i: (0, 0)),
        ],
        out_specs=[
            pl.BlockSpec((TILE, 1), lambda i: (i, 0)),
            pl.BlockSpec((TILE, 1), lambda i: (i, 0)),
            pl.BlockSpec((TILE, 1), lambda i: (i, 0)),
            pl.BlockSpec((TILE, 1), lambda i: (i, 0)),
            pl.BlockSpec((TILE, 1), lambda i: (i, 0)),
            pl.BlockSpec((TILE, 1), lambda i: (i, 0)),
            pl.BlockSpec((1, E), lambda i: (0, 0)),
            pl.BlockSpec((128, 1), lambda i: (0, 0)),
            pl.BlockSpec((128, 1), lambda i: (0, 0)),
            pl.BlockSpec((128, 1), lambda i: (0, 0)),
        ],
        out_shape=[
            jax.ShapeDtypeStruct((T, 1), jnp.int32),
            jax.ShapeDtypeStruct((T, 1), jnp.int32),
            jax.ShapeDtypeStruct((T, 1), jnp.int32),
            jax.ShapeDtypeStruct((T, 1), jnp.int32),
            jax.ShapeDtypeStruct((T, 1), jnp.float32),
            jax.ShapeDtypeStruct((T, 1), jnp.float32),
            jax.ShapeDtypeStruct((1, E), jnp.int32),
            jax.ShapeDtypeStruct((128, 1), jnp.int32),
            jax.ShapeDtypeStruct((128, 1), jnp.int32),
            jax.ShapeDtypeStruct((128, 1), jnp.int32),
        ],
        scratch_shapes=[pltpu.VMEM((1, E), jnp.float32)],
        compiler_params=pltpu.CompilerParams(
            dimension_semantics=("arbitrary",),
        ),
    )(x_flat, Wg, bg2)


_NBUF = 4


def _dispatch_body(x_hbm, i1_hbm, i2_hbm, r1_hbm, r2_hbm, seg_hbm,
                   xs_hbm, d1_hbm, d2_hbm,
                   i1_v, i2_v, r1_v, r2_v, seg_v, d1_v, d2_v, xbufs,
                   gsem, ssem):
    wid = lax.axis_index("s") * NC + lax.axis_index("c")
    base = wid * TPW
    pltpu.sync_copy(i1_hbm.at[pl.ds(base, TPW)], i1_v)
    pltpu.sync_copy(i2_hbm.at[pl.ds(base, TPW)], i2_v)
    pltpu.sync_copy(r1_hbm.at[pl.ds(base, TPW)], r1_v)
    pltpu.sync_copy(r2_hbm.at[pl.ds(base, TPW)], r2_v)
    pltpu.sync_copy(seg_hbm, seg_v)
    dests = []
    for c in range(NCH):
        i1c = i1_v[pl.ds(c * NL, NL)]
        i2c = i2_v[pl.ds(c * NL, NL)]
        d1 = plsc.load_gather(seg_v, [i1c]) + r1_v[pl.ds(c * NL, NL)]
        d2 = plsc.load_gather(seg_v, [i2c]) + r2_v[pl.ds(c * NL, NL)]
        d1_v[pl.ds(c * NL, NL)] = d1
        d2_v[pl.ds(c * NL, NL)] = d2
        dests.append((d1, d2))
    cpd1 = pltpu.async_copy(d1_v, d1_hbm.at[pl.ds(base, TPW)], ssem)
    cpd2 = pltpu.async_copy(d2_v, d2_hbm.at[pl.ds(base, TPW)], ssem)
    gcp = [None] * NCH
    scp = [None] * NCH
    for c in range(min(_NBUF, NCH)):
        gcp[c] = pltpu.async_copy(
            x_hbm.at[pl.ds(base + c * NL, NL), :], xbufs.at[c % _NBUF], gsem)
    for c in range(NCH):
        if c >= 1 and c + _NBUF - 1 < NCH:
            scp[c - 1][0].wait()
            scp[c - 1][1].wait()
            nxt = c + _NBUF - 1
            gcp[nxt] = pltpu.async_copy(
                x_hbm.at[pl.ds(base + nxt * NL, NL), :],
                xbufs.at[nxt % _NBUF], gsem)
        gcp[c].wait()
        d1, d2 = dests[c]
        s1 = pltpu.async_copy(xbufs.at[c % _NBUF], xs_hbm.at[d1], ssem)
        s2 = pltpu.async_copy(xbufs.at[c % _NBUF], xs_hbm.at[d2], ssem)
        scp[c] = (s1, s2)
    for c in range(max(0, NCH - _NBUF), NCH):
        scp[c][0].wait()
        scp[c][1].wait()
    cpd1.wait()
    cpd2.wait()


def _dispatch_call(x_flat, i1, i2, r1, r2, seg16):
    mesh = plsc.VectorSubcoreMesh(core_axis_name="c", subcore_axis_name="s")
    return pl.kernel(
        _dispatch_body,
        out_type=[
            jax.ShapeDtypeStruct((PADT, D), jnp.float32),
            jax.ShapeDtypeStruct((T,), jnp.int32),
            jax.ShapeDtypeStruct((T,), jnp.int32),
        ],
        mesh=mesh,
        scratch_types=[
            pltpu.VMEM((TPW,), jnp.int32),
            pltpu.VMEM((TPW,), jnp.int32),
            pltpu.VMEM((TPW,), jnp.int32),
            pltpu.VMEM((TPW,), jnp.int32),
            pltpu.VMEM((NL,), jnp.int32),
            pltpu.VMEM((TPW,), jnp.int32),
            pltpu.VMEM((TPW,), jnp.int32),
            pltpu.VMEM((_NBUF, NL, D), jnp.float32),
            pltpu.SemaphoreType.DMA,
            pltpu.SemaphoreType.DMA,
        ],
        compiler_params=pltpu.CompilerParams(needs_layout_passes=False),
    )(x_flat, i1, i2, r1, r2, seg16)


def _ffn_body(eot_ref, nxt_ref, slt_ref, x_ref, w1_hbm, b1_ref, w2_hbm,
              b2_ref, y_ref, wb1, wb2, sem1, sem2):
    i = pl.program_id(0)
    e = eot_ref[i]
    sl = slt_ref[i]
    prev = eot_ref[jnp.maximum(i - 1, 0)]
    first = jnp.logical_or(i == 0, e != prev)

    def w_copies(ee, ss):
        return (pltpu.make_async_copy(w1_hbm.at[ee], wb1.at[ss], sem1.at[ss]),
                pltpu.make_async_copy(w2_hbm.at[ee], wb2.at[ss], sem2.at[ss]))

    @pl.when(i == 0)
    def _():
        c1, c2 = w_copies(e, sl)
        c1.start()
        c2.start()

    @pl.when(first)
    def _():
        en = nxt_ref[i]

        @pl.when(en < E)
        def _():
            c1, c2 = w_copies(en, 1 - sl)
            c1.start()
            c2.start()

        c1, c2 = w_copies(e, sl)
        c1.wait()
        c2.wait()

    x = x_ref[...]
    h = jnp.dot(x, wb1[sl], preferred_element_type=jnp.float32) + b1_ref[0]
    h = jnp.maximum(h, 0.0)
    y = jnp.dot(h, wb2[sl], preferred_element_type=jnp.float32) + b2_ref[0]
    y_ref[...] = y


def _ffn_call(eot, nxt, slt, xs, W1, b1r, W2, b2r):
    grid_spec = pltpu.PrefetchScalarGridSpec(
        num_scalar_prefetch=3,
        grid=(NT,),
        in_specs=[
            pl.BlockSpec((TILE, D), lambda i, eot, nxt, slt: (i, 0)),
            pl.BlockSpec(memory_space=pl.ANY),
            pl.BlockSpec((1, 1, DFF), lambda i, eot, nxt, slt: (eot[i], 0, 0)),
            pl.BlockSpec(memory_space=pl.ANY),
            pl.BlockSpec((1, 1, D), lambda i, eot, nxt, slt: (eot[i], 0, 0)),
        ],
        out_specs=pl.BlockSpec((TILE, D), lambda i, eot, nxt, slt: (i, 0)),
        scratch_shapes=[
            pltpu.VMEM((2, D, DFF), jnp.float32),
            pltpu.VMEM((2, DFF, D), jnp.float32),
            pltpu.SemaphoreType.DMA((2,)),
            pltpu.SemaphoreType.DMA((2,)),
        ],
    )
    return pl.pallas_call(
        _ffn_body,
        grid_spec=grid_spec,
        out_shape=jax.ShapeDtypeStruct((PADT, D), jnp.float32),
        compiler_params=pltpu.CompilerParams(
            dimension_semantics=("arbitrary",),
        ),
    )(eot, nxt, slt, xs, W1, b1r, W2, b2r)


def _combine_body(y_hbm, d1_hbm, d2_hbm, g1_hbm, g2_hbm, out_hbm,
                  d1_v, d2_v, g1_v, g2_v, bufa, bufb, obuf, gsem, osem):
    wid = lax.axis_index("s") * NC + lax.axis_index("c")
    base = wid * TPW
    pltpu.sync_copy(d1_hbm.at[pl.ds(base, TPW)], d1_v)
    pltpu.sync_copy(d2_hbm.at[pl.ds(base, TPW)], d2_v)
    pltpu.sync_copy(g1_hbm.at[pl.ds(base, TPW)], g1_v)
    pltpu.sync_copy(g2_hbm.at[pl.ds(base, TPW)], g2_v)

    def issue_gathers(c):
        idx1 = d1_v[pl.ds(c * NL, NL)]
        idx2 = d2_v[pl.ds(c * NL, NL)]
        cp1 = pltpu.async_copy(y_hbm.at[idx1], bufa.at[c % 2], gsem)
        cp2 = pltpu.async_copy(y_hbm.at[idx2], bufb.at[c % 2], gsem)
        return cp1, cp2

    gcp = [None] * NCH
    ocp = [None] * NCH
    gcp[0] = issue_gathers(0)
    gcp[1] = issue_gathers(1)
    for c in range(NCH):
        gcp[c][0].wait()
        gcp[c][1].wait()
        if c >= 2:
            ocp[c - 2].wait()
        ba = bufa.at[c % 2]
        bb = bufb.at[c % 2]
        ob = obuf.at[c % 2]

        def row_body(r, _):
            ridx = jnp.full((NL,), c * NL + r, jnp.int32)
            ga = plsc.load_gather(g1_v, [ridx])
            gb = plsc.load_gather(g2_v, [ridx])
            for k in range(D // NL):
                a = ba[r, pl.ds(k * NL, NL)]
                b = bb[r, pl.ds(k * NL, NL)]
                ob[r, pl.ds(k * NL, NL)] = a * ga + b * gb
            return 0

        lax.fori_loop(0, NL, row_body, 0)
        if c + 2 < NCH:
            gcp[c + 2] = issue_gathers(c + 2)
        ocp[c] = pltpu.async_copy(ob, out_hbm.at[pl.ds(base + c * NL, NL), :],
                                  osem)
    ocp[NCH - 2].wait()
    ocp[NCH - 1].wait()


def _combine_call(ys, d1, d2, g1, g2):
    mesh = plsc.VectorSubcoreMesh(core_axis_name="c", subcore_axis_name="s")
    return pl.kernel(
        _combine_body,
        out_type=jax.ShapeDtypeStruct((T, D), jnp.float32),
        mesh=mesh,
        scratch_types=[
            pltpu.VMEM((TPW,), jnp.int32),
            pltpu.VMEM((TPW,), jnp.int32),
            pltpu.VMEM((TPW,), jnp.float32),
            pltpu.VMEM((TPW,), jnp.float32),
            pltpu.VMEM((2, NL, D), jnp.float32),
            pltpu.VMEM((2, NL, D), jnp.float32),
            pltpu.VMEM((2, NL, D), jnp.float32),
            pltpu.SemaphoreType.DMA,
            pltpu.SemaphoreType.DMA,
        ],
        compiler_params=pltpu.CompilerParams(needs_layout_passes=False),
    )(ys, d1, d2, g1, g2)


def kernel(x, W1, b1, W2, b2, Wg, bg):
    x_flat = x.reshape(T, D)
    (i1, i2, r1, r2, g1, g2, seg, eot, nxt, slt) = _router_call(
        x_flat, Wg, bg.reshape(1, E))
    i1 = i1.reshape(T)
    i2 = i2.reshape(T)
    r1 = r1.reshape(T)
    r2 = r2.reshape(T)
    g1 = g1.reshape(T)
    g2 = g2.reshape(T)
    seg16 = jnp.pad(seg.reshape(E), (0, NL - E))
    eot = eot.reshape(128)[:NT]
    nxt = nxt.reshape(128)[:NT]
    slt = slt.reshape(128)[:NT]
    xs, d1, d2 = _dispatch_call(x_flat, i1, i2, r1, r2, seg16)
    ys = _ffn_call(eot, nxt, slt, xs, W1, b1.reshape(E, 1, DFF),
                   W2, b2.reshape(E, 1, D))
    out = _combine_call(ys, d1, d2, g1, g2)
    return out.reshape(B, N, D)


# speedup vs baseline: 2.6390x; 1.0397x over previous
import jax
import jax.numpy as jnp
from jax import lax
from jax.experimental import pallas as pl
from jax.experimental.pallas import tpu as pltpu
from jax.experimental.pallas import tpu_sc as plsc

B, N, D = 2, 2048, 1024
E, TOPK, DFF = 8, 2, 2 * 1024
T = B * N
TILE = 256
NT = T * TOPK // TILE + (E - 1)
PADT = NT * TILE
NC, NS, NL = 2, 16, 16
NW = NC * NS
TPW = T // NW
NCH = TPW // NL
DI = D // 2


def _router_body(x_ref, wg_ref, bg_ref,
                 i1_ref, i2_ref, r1_ref, r2_ref, g1_ref, g2_ref,
                 seg_ref, eot_ref, nxt_ref, slt_ref, lc_ref, carry_ref):
    step = pl.program_id(0)

    @pl.when(step == 0)
    def _():
        carry_ref[...] = jnp.zeros_like(carry_ref)

    x = x_ref[...]
    logits = jnp.dot(x, wg_ref[...], preferred_element_type=jnp.float32) + bg_ref[...]
    col = lax.broadcasted_iota(jnp.int32, logits.shape, 1)
    l1 = jnp.max(logits, axis=1, keepdims=True)
    i1 = jnp.min(jnp.where(logits >= l1, col, E), axis=1, keepdims=True)
    masked = jnp.where(col == i1, -jnp.inf, logits)
    l2 = jnp.max(masked, axis=1, keepdims=True)
    i2 = jnp.min(jnp.where(masked >= l2, col, E), axis=1, keepdims=True)
    e2 = jnp.exp(l2 - l1)
    g1 = 1.0 / (1.0 + e2)
    g2 = e2 / (1.0 + e2)

    oh1 = (col == i1).astype(jnp.float32)
    oh2 = (col == i2).astype(jnp.float32)
    oh = oh1 + oh2
    r_i = lax.broadcasted_iota(jnp.int32, (TILE, TILE), 0)
    c_i = lax.broadcasted_iota(jnp.int32, (TILE, TILE), 1)
    tril = (r_i > c_i).astype(jnp.float32)
    carry = carry_ref[...]
    pre = jnp.dot(tril, oh, preferred_element_type=jnp.float32) + carry
    rank1 = jnp.sum(pre * oh1, axis=1, keepdims=True)
    rank2 = jnp.sum(pre * oh2, axis=1, keepdims=True)
    counts = carry + jnp.sum(oh, axis=0, keepdims=True)
    carry_ref[...] = counts

    i1_ref[...] = i1
    i2_ref[...] = i2
    r1_ref[...] = rank1.astype(jnp.int32)
    r2_ref[...] = rank2.astype(jnp.int32)
    g1_ref[...] = g1
    g2_ref[...] = g2

    tiles_i = (counts.astype(jnp.int32) + (TILE - 1)) // TILE
    triu = (r_i[:E, :E] < c_i[:E, :E]).astype(jnp.float32)
    base_t = jnp.dot(tiles_i.astype(jnp.float32), triu,
                     preferred_element_type=jnp.float32).astype(jnp.int32)
    seg_ref[...] = base_t * TILE
    ti = lax.broadcasted_iota(jnp.int32, (128, E), 0)
    ge = (ti >= base_t).astype(jnp.int32)
    eot_ref[...] = jnp.sum(ge, axis=1, keepdims=True) - 1
    ecol = lax.broadcasted_iota(jnp.int32, (128, E), 1)
    nz = (tiles_i > 0)
    nxt_ref[...] = jnp.min(
        jnp.where(nz & (base_t > ti), ecol, E), axis=1, keepdims=True)
    ap = jnp.sum((nz & (base_t <= ti)).astype(jnp.int32), axis=1, keepdims=True)
    slt_ref[...] = (ap - 1) & 1
    lc_ref[...] = base_t + tiles_i


def _router_call(x_flat, Wg, bg2):
    n_blocks = T // TILE
    return pl.pallas_call(
        _router_body,
        grid=(n_blocks,),
        in_specs=[
            pl.BlockSpec((TILE, D), lambda i: (i, 0)),
            pl.BlockSpec((D, E), lambda i: (0, 0)),
            pl.BlockSpec((1, E), lambda i: (0, 0)),
        ],
        out_specs=[
            pl.BlockSpec((TILE, 1), lambda i: (i, 0)),
            pl.BlockSpec((TILE, 1), lambda i: (i, 0)),
            pl.BlockSpec((TILE, 1), lambda i: (i, 0)),
            pl.BlockSpec((TILE, 1), lambda i: (i, 0)),
            pl.BlockSpec((TILE, 1), lambda i: (i, 0)),
            pl.BlockSpec((TILE, 1), lambda i: (i, 0)),
            pl.BlockSpec((1, E), lambda i: (0, 0)),
            pl.BlockSpec((128, 1), lambda i: (0, 0)),
            pl.BlockSpec((128, 1), lambda i: (0, 0)),
            pl.BlockSpec((128, 1), lambda i: (0, 0)),
            pl.BlockSpec((1, E), lambda i: (0, 0)),
        ],
        out_shape=[
            jax.ShapeDtypeStruct((T, 1), jnp.int32),
            jax.ShapeDtypeStruct((T, 1), jnp.int32),
            jax.ShapeDtypeStruct((T, 1), jnp.int32),
            jax.ShapeDtypeStruct((T, 1), jnp.int32),
            jax.ShapeDtypeStruct((T, 1), jnp.float32),
            jax.ShapeDtypeStruct((T, 1), jnp.float32),
            jax.ShapeDtypeStruct((1, E), jnp.int32),
            jax.ShapeDtypeStruct((128, 1), jnp.int32),
            jax.ShapeDtypeStruct((128, 1), jnp.int32),
            jax.ShapeDtypeStruct((128, 1), jnp.int32),
            jax.ShapeDtypeStruct((1, E), jnp.int32),
        ],
        scratch_shapes=[pltpu.VMEM((1, E), jnp.float32)],
        compiler_params=pltpu.CompilerParams(
            dimension_semantics=("arbitrary",),
        ),
    )(x_flat, Wg, bg2)


_NBUF = 4


def _dispatch_body(x_hbm, i1_hbm, i2_hbm, r1_hbm, r2_hbm, seg_hbm,
                   xs_hbm, d1_hbm, d2_hbm,
                   i1_v, i2_v, r1_v, r2_v, seg_v, d1_v, d2_v, xbufs,
                   gsem, ssem):
    wid = lax.axis_index("s") * NC + lax.axis_index("c")
    base = wid * TPW
    pltpu.sync_copy(i1_hbm.at[pl.ds(base, TPW)], i1_v)
    pltpu.sync_copy(i2_hbm.at[pl.ds(base, TPW)], i2_v)
    pltpu.sync_copy(r1_hbm.at[pl.ds(base, TPW)], r1_v)
    pltpu.sync_copy(r2_hbm.at[pl.ds(base, TPW)], r2_v)
    pltpu.sync_copy(seg_hbm, seg_v)
    dests = []
    for c in range(NCH):
        i1c = i1_v[pl.ds(c * NL, NL)]
        i2c = i2_v[pl.ds(c * NL, NL)]
        d1 = plsc.load_gather(seg_v, [i1c]) + r1_v[pl.ds(c * NL, NL)]
        d2 = plsc.load_gather(seg_v, [i2c]) + r2_v[pl.ds(c * NL, NL)]
        d1_v[pl.ds(c * NL, NL)] = d1
        d2_v[pl.ds(c * NL, NL)] = d2
        dests.append((d1, d2))
    cpd1 = pltpu.async_copy(d1_v, d1_hbm.at[pl.ds(base, TPW)], ssem)
    cpd2 = pltpu.async_copy(d2_v, d2_hbm.at[pl.ds(base, TPW)], ssem)
    gcp = [None] * NCH
    scp = [None] * NCH
    for c in range(min(_NBUF, NCH)):
        gcp[c] = pltpu.async_copy(
            x_hbm.at[pl.ds(base + c * NL, NL), :], xbufs.at[c % _NBUF], gsem)
    for c in range(NCH):
        if c >= 1 and c + _NBUF - 1 < NCH:
            scp[c - 1][0].wait()
            scp[c - 1][1].wait()
            nxt = c + _NBUF - 1
            gcp[nxt] = pltpu.async_copy(
                x_hbm.at[pl.ds(base + nxt * NL, NL), :],
                xbufs.at[nxt % _NBUF], gsem)
        gcp[c].wait()
        d1, d2 = dests[c]
        s1 = pltpu.async_copy(xbufs.at[c % _NBUF], xs_hbm.at[d1], ssem)
        s2 = pltpu.async_copy(xbufs.at[c % _NBUF], xs_hbm.at[d2], ssem)
        scp[c] = (s1, s2)
    for c in range(max(0, NCH - _NBUF), NCH):
        scp[c][0].wait()
        scp[c][1].wait()
    cpd1.wait()
    cpd2.wait()


def _dispatch_call(x_flat, i1, i2, r1, r2, seg16):
    mesh = plsc.VectorSubcoreMesh(core_axis_name="c", subcore_axis_name="s")
    return pl.kernel(
        _dispatch_body,
        out_type=[
            jax.ShapeDtypeStruct((PADT, D), jnp.float32),
            jax.ShapeDtypeStruct((T,), jnp.int32),
            jax.ShapeDtypeStruct((T,), jnp.int32),
        ],
        mesh=mesh,
        scratch_types=[
            pltpu.VMEM((TPW,), jnp.int32),
            pltpu.VMEM((TPW,), jnp.int32),
            pltpu.VMEM((TPW,), jnp.int32),
            pltpu.VMEM((TPW,), jnp.int32),
            pltpu.VMEM((NL,), jnp.int32),
            pltpu.VMEM((TPW,), jnp.int32),
            pltpu.VMEM((TPW,), jnp.int32),
            pltpu.VMEM((_NBUF, NL, D), jnp.float32),
            pltpu.SemaphoreType.DMA,
            pltpu.SemaphoreType.DMA,
        ],
        compiler_params=pltpu.CompilerParams(needs_layout_passes=False),
    )(x_flat, i1, i2, r1, r2, seg16)


def _ffn_body(eot_ref, nxt_ref, slt_ref, ntl_ref, x_ref, w1_hbm, b1_ref,
              w2_hbm, b2_ref, y_ref, wb1, wb2, sem1, sem2):
    i = pl.program_id(0)
    e = eot_ref[i]
    sl = slt_ref[i]
    prev = eot_ref[jnp.maximum(i - 1, 0)]
    first = jnp.logical_or(i == 0, e != prev)

    def w_copies(ee, ss):
        return (pltpu.make_async_copy(w1_hbm.at[ee], wb1.at[ss], sem1.at[ss]),
                pltpu.make_async_copy(w2_hbm.at[ee], wb2.at[ss], sem2.at[ss]))

    @pl.when(i == 0)
    def _():
        c1, c2 = w_copies(e, sl)
        c1.start()
        c2.start()

    @pl.when(first)
    def _():
        en = nxt_ref[i]

        @pl.when(en < E)
        def _():
            c1, c2 = w_copies(en, 1 - sl)
            c1.start()
            c2.start()

        c1, c2 = w_copies(e, sl)
        c1.wait()
        c2.wait()

    @pl.when(i < ntl_ref[0])
    def _():
        x = x_ref[...]
        h = jnp.dot(x, wb1[sl], preferred_element_type=jnp.float32) + b1_ref[0]
        h = jnp.maximum(h, 0.0)
        y = jnp.dot(h, wb2[sl], preferred_element_type=jnp.float32) + b2_ref[0]
        y_ref[...] = y


def _ffn_call(eot, nxt, slt, ntl, xs, W1, b1r, W2, b2r):
    grid_spec = pltpu.PrefetchScalarGridSpec(
        num_scalar_prefetch=4,
        grid=(NT,),
        in_specs=[
            pl.BlockSpec((TILE, D),
                         lambda i, eot, nxt, slt, ntl:
                         (jnp.where(i < ntl[0], i, 0), 0)),
            pl.BlockSpec(memory_space=pl.ANY),
            pl.BlockSpec((1, 1, DFF),
                         lambda i, eot, nxt, slt, ntl: (eot[i], 0, 0)),
            pl.BlockSpec(memory_space=pl.ANY),
            pl.BlockSpec((1, 1, D),
                         lambda i, eot, nxt, slt, ntl: (eot[i], 0, 0)),
        ],
        out_specs=pl.BlockSpec((TILE, D),
                               lambda i, eot, nxt, slt, ntl: (i, 0)),
        scratch_shapes=[
            pltpu.VMEM((2, D, DFF), jnp.float32),
            pltpu.VMEM((2, DFF, D), jnp.float32),
            pltpu.SemaphoreType.DMA((2,)),
            pltpu.SemaphoreType.DMA((2,)),
        ],
    )
    return pl.pallas_call(
        _ffn_body,
        grid_spec=grid_spec,
        out_shape=jax.ShapeDtypeStruct((PADT, D), jnp.float32),
        compiler_params=pltpu.CompilerParams(
            dimension_semantics=("arbitrary",),
        ),
    )(eot, nxt, slt, ntl, xs, W1, b1r, W2, b2r)


def _combine_body(y_hbm, d1_hbm, d2_hbm, g1_hbm, g2_hbm, out_hbm,
                  d1_v, d2_v, g1_v, g2_v, bufa, bufb, obuf, gsem, osem):
    wid = lax.axis_index("s") * NC + lax.axis_index("c")
    base = wid * TPW
    pltpu.sync_copy(d1_hbm.at[pl.ds(base, TPW)], d1_v)
    pltpu.sync_copy(d2_hbm.at[pl.ds(base, TPW)], d2_v)
    pltpu.sync_copy(g1_hbm.at[pl.ds(base, TPW)], g1_v)
    pltpu.sync_copy(g2_hbm.at[pl.ds(base, TPW)], g2_v)

    def issue_gathers(c):
        idx1 = d1_v[pl.ds(c * NL, NL)]
        idx2 = d2_v[pl.ds(c * NL, NL)]
        cp1 = pltpu.async_copy(y_hbm.at[idx1], bufa.at[c % 2], gsem)
        cp2 = pltpu.async_copy(y_hbm.at[idx2], bufb.at[c % 2], gsem)
        return cp1, cp2

    gcp = [None] * NCH
    ocp = [None] * NCH
    gcp[0] = issue_gathers(0)
    gcp[1] = issue_gathers(1)
    for c in range(NCH):
        gcp[c][0].wait()
        gcp[c][1].wait()
        if c >= 2:
            ocp[c - 2].wait()
        ba = bufa.at[c % 2]
        bb = bufb.at[c % 2]
        ob = obuf.at[c % 2]

        def row_body(r, _):
            ridx = jnp.full((NL,), c * NL + r, jnp.int32)
            ga = plsc.load_gather(g1_v, [ridx])
            gb = plsc.load_gather(g2_v, [ridx])
            for k in range(D // NL):
                a = ba[r, pl.ds(k * NL, NL)]
                b = bb[r, pl.ds(k * NL, NL)]
                ob[r, pl.ds(k * NL, NL)] = a * ga + b * gb
            return 0

        lax.fori_loop(0, NL, row_body, 0)
        if c + 2 < NCH:
            gcp[c + 2] = issue_gathers(c + 2)
        ocp[c] = pltpu.async_copy(ob, out_hbm.at[pl.ds(base + c * NL, NL), :],
                                  osem)
    ocp[NCH - 2].wait()
    ocp[NCH - 1].wait()


def _combine_call(ys, d1, d2, g1, g2):
    mesh = plsc.VectorSubcoreMesh(core_axis_name="c", subcore_axis_name="s")
    return pl.kernel(
        _combine_body,
        out_type=jax.ShapeDtypeStruct((T, D), jnp.float32),
        mesh=mesh,
        scratch_types=[
            pltpu.VMEM((TPW,), jnp.int32),
            pltpu.VMEM((TPW,), jnp.int32),
            pltpu.VMEM((TPW,), jnp.float32),
            pltpu.VMEM((TPW,), jnp.float32),
            pltpu.VMEM((2, NL, D), jnp.float32),
            pltpu.VMEM((2, NL, D), jnp.float32),
            pltpu.VMEM((2, NL, D), jnp.float32),
            pltpu.SemaphoreType.DMA,
            pltpu.SemaphoreType.DMA,
        ],
        compiler_params=pltpu.CompilerParams(needs_layout_passes=False),
    )(ys, d1, d2, g1, g2)


def kernel(x, W1, b1, W2, b2, Wg, bg):
    x_flat = x.reshape(T, D)
    (i1, i2, r1, r2, g1, g2, seg, eot, nxt, slt, lc) = _router_call(
        x_flat, Wg, bg.reshape(1, E))
    i1 = i1.reshape(T)
    i2 = i2.reshape(T)
    r1 = r1.reshape(T)
    r2 = r2.reshape(T)
    g1 = g1.reshape(T)
    g2 = g2.reshape(T)
    seg16 = jnp.pad(seg.reshape(E), (0, NL - E))
    eot = eot.reshape(128)[:NT]
    nxt = nxt.reshape(128)[:NT]
    slt = slt.reshape(128)[:NT]
    ntl = lc.reshape(E)[E - 1:]
    xs, d1, d2 = _dispatch_call(x_flat, i1, i2, r1, r2, seg16)
    ys = _ffn_call(eot, nxt, slt, ntl, xs, W1, b1.reshape(E, 1, DFF),
                   W2, b2.reshape(E, 1, D))
    out = _combine_call(ys, d1, d2, g1, g2)
    return out.reshape(B, N, D)
